# Initial kernel scaffold; baseline (speedup 1.0000x reference)
import jax
import jax.numpy as jnp
from jax.experimental import pallas as pl


def kernel(Z, position, atomic_masses):
    raise NotImplementedError("write your pallas kernel here")


# speedup vs baseline: 1.3920x; 1.3920x over previous
import functools

import jax
import jax.numpy as jnp
from jax import lax
from jax.experimental import pallas as pl
from jax.experimental.pallas import tpu as pltpu
from jax.experimental.pallas import tpu_sc as plsc

NC, NS, L = 2, 16, 16
NW = NC * NS
N = 1_000_000
C = 8_000
NCHUNK = N // C
GRP = C // L
TRI = 3 * C // (3 * L)
FULL = NCHUNK // NW
REM = NCHUNK % NW

_mesh = plsc.VectorSubcoreMesh(core_axis_name="c", subcore_axis_name="s")
_params = pltpu.CompilerParams(needs_layout_passes=False)


def _wid():
    return lax.axis_index("s") * NC + lax.axis_index("c")


@functools.partial(
    pl.kernel,
    out_type=jax.ShapeDtypeStruct((NW * 4 * L,), jnp.float32),
    mesh=_mesh,
    compiler_params=_params,
    scratch_types=[
        pltpu.VMEM((128,), jnp.float32),
        pltpu.VMEM((C,), jnp.int32),
        pltpu.VMEM((3 * C,), jnp.float32),
        pltpu.VMEM((4 * L,), jnp.float32),
    ],
)
def _partial_sums(z_hbm, pos_hbm, tab_hbm, part_hbm, tab_v, z_v, p_v, acc_v):
    wid = _wid()
    pltpu.sync_copy(tab_hbm, tab_v)
    i3 = lax.iota(jnp.int32, L) * 3
    zero = jnp.zeros((L,), jnp.float32)
    nc = FULL + jnp.where(wid < REM, 1, 0)

    def chunk_body(i, carry):
        c = wid + i * NW
        pltpu.sync_copy(z_hbm.at[pl.ds(c * C, C)], z_v)
        pltpu.sync_copy(pos_hbm.at[pl.ds(c * (3 * C), 3 * C)], p_v)

        def grp_body(g, acc):
            ax, ay, az, am = acc
            z = z_v[pl.ds(g * L, L)]
            m = plsc.load_gather(tab_v, [z])
            base = g * (3 * L) + i3
            px = plsc.load_gather(p_v, [base])
            py = plsc.load_gather(p_v, [base + 1])
            pz = plsc.load_gather(p_v, [base + 2])
            return (ax + m * px, ay + m * py, az + m * pz, am + m)

        return lax.fori_loop(0, GRP, grp_body, carry)

    ax, ay, az, am = lax.fori_loop(0, nc, chunk_body, (zero, zero, zero, zero))
    acc_v[pl.ds(0, L)] = ax
    acc_v[pl.ds(L, L)] = ay
    acc_v[pl.ds(2 * L, L)] = az
    acc_v[pl.ds(3 * L, L)] = am
    pltpu.sync_copy(acc_v, part_hbm.at[pl.ds(wid * (4 * L), 4 * L)])


@functools.partial(
    pl.kernel,
    out_type=jax.ShapeDtypeStruct((3 * N,), jnp.float32),
    mesh=_mesh,
    compiler_params=_params,
    scratch_types=[
        pltpu.VMEM((NW * 4 * L,), jnp.float32),
        pltpu.VMEM((3 * C,), jnp.float32),
    ],
)
def _subtract(pos_hbm, part_hbm, out_hbm, part_v, p_v):
    wid = _wid()
    pltpu.sync_copy(part_hbm, part_v)
    zero = jnp.zeros((L,), jnp.float32)

    def red(w, acc):
        ax, ay, az, am = acc
        b = w * (4 * L)
        return (
            ax + part_v[pl.ds(b, L)],
            ay + part_v[pl.ds(b + L, L)],
            az + part_v[pl.ds(b + 2 * L, L)],
            am + part_v[pl.ds(b + 3 * L, L)],
        )

    ax, ay, az, am = lax.fori_loop(0, NW, red, (zero, zero, zero, zero))
    sm = jnp.sum(am)
    sx = jnp.sum(ax)
    sy = jnp.sum(ay)
    sz = jnp.sum(az)
    smv = jnp.full((L,), sm, jnp.float32)
    iota = lax.iota(jnp.int32, L)
    coms = []
    for p in range(3):
        r = (iota + p * L) % 3
        numer = jnp.where(r == 0, sx, jnp.where(r == 1, sy, sz))
        coms.append(numer / smv)
    nc = FULL + jnp.where(wid < REM, 1, 0)

    def chunk_body(i, carry):
        c = wid + i * NW
        pltpu.sync_copy(pos_hbm.at[pl.ds(c * (3 * C), 3 * C)], p_v)

        def tri_body(t, carry2):
            b = t * (3 * L)
            for p in range(3):
                off = b + p * L
                p_v[pl.ds(off, L)] = p_v[pl.ds(off, L)] - coms[p]
            return carry2

        lax.fori_loop(0, TRI, tri_body, 0)
        pltpu.sync_copy(p_v, out_hbm.at[pl.ds(c * (3 * C), 3 * C)])
        return carry

    lax.fori_loop(0, nc, chunk_body, 0)


def kernel(Z, position, atomic_masses):
    posf = position.reshape(-1)
    tab = jnp.zeros((128,), jnp.float32).at[: atomic_masses.shape[0]].set(atomic_masses)
    parts = _partial_sums(Z, posf, tab)
    outf = _subtract(posf, parts)
    return outf.reshape(position.shape)


# speedup vs baseline: 69.1005x; 49.6406x over previous
import functools

import jax
import jax.numpy as jnp
from jax import lax
from jax.experimental import pallas as pl
from jax.experimental.pallas import tpu as pltpu
from jax.experimental.pallas import tpu_sc as plsc

NC, NS, L = 2, 16, 16
NW = NC * NS
N = 1_000_000
C = 8_000
NCHUNK = N // C
GRP = C // L
FULL = NCHUNK // NW
REM = NCHUNK % NW

_mesh = plsc.VectorSubcoreMesh(core_axis_name="c", subcore_axis_name="s")
_params = pltpu.CompilerParams(needs_layout_passes=False)


@functools.partial(
    pl.kernel,
    out_type=jax.ShapeDtypeStruct((N,), jnp.float32),
    mesh=_mesh,
    compiler_params=_params,
    scratch_types=[
        pltpu.VMEM((128,), jnp.float32),
        pltpu.VMEM((C,), jnp.int32),
        pltpu.VMEM((C,), jnp.float32),
    ],
)
def _sc_gather_masses(z_hbm, tab_hbm, m_hbm, tab_v, z_v, m_v):
    wid = lax.axis_index("s") * NC + lax.axis_index("c")
    pltpu.sync_copy(tab_hbm, tab_v)
    nc = FULL + jnp.where(wid < REM, 1, 0)

    def chunk_body(i, carry):
        c = wid + i * NW
        pltpu.sync_copy(z_hbm.at[pl.ds(c * C, C)], z_v)

        def grp_body(g, carry2):
            z = z_v[pl.ds(g * L, L)]
            m_v[pl.ds(g * L, L)] = plsc.load_gather(tab_v, [z])
            return carry2

        lax.fori_loop(0, GRP, grp_body, 0)
        pltpu.sync_copy(m_v, m_hbm.at[pl.ds(c * C, C)])
        return carry

    lax.fori_loop(0, nc, chunk_body, 0)


BR = 32_768
NBR = -(-N // BR)


def _tc_reduce_body(pos_ref, m_ref, out_ref, acc_ref):
    i = pl.program_id(0)

    @pl.when(i == 0)
    def _():
        acc_ref[...] = jnp.zeros_like(acc_ref)

    lane = lax.broadcasted_iota(jnp.int32, (1, BR), 1) + i * BR
    m = jnp.where(lane < N, m_ref[...].reshape(1, BR), 0.0)
    p = pos_ref[...]
    acc_ref[0:3, :] += m * p
    acc_ref[3:4, :] += m

    @pl.when(i == NBR - 1)
    def _():
        out_ref[...] = jnp.broadcast_to(
            jnp.sum(acc_ref[...], axis=1, keepdims=True), (4, 128)
        )


_tc_reduce = pl.pallas_call(
    _tc_reduce_body,
    grid=(NBR,),
    in_specs=[
        pl.BlockSpec((3, BR), lambda i: (0, i)),
        pl.BlockSpec((BR,), lambda i: (i,)),
    ],
    out_specs=pl.BlockSpec((4, 128), lambda i: (0, 0)),
    out_shape=jax.ShapeDtypeStruct((4, 128), jnp.float32),
    scratch_shapes=[pltpu.VMEM((4, BR), jnp.float32)],
)

BS = 65_536
NBS = -(-N // BS)


def _tc_subtract_body(pos_ref, sums_ref, out_ref):
    s = sums_ref[...]
    com = s[0:3, 0:1] / s[3:4, 0:1]
    out_ref[...] = pos_ref[...] - com


_tc_subtract = pl.pallas_call(
    _tc_subtract_body,
    grid=(NBS,),
    in_specs=[
        pl.BlockSpec((3, BS), lambda i: (0, i)),
        pl.BlockSpec((4, 128), lambda i: (0, 0)),
    ],
    out_specs=pl.BlockSpec((3, BS), lambda i: (0, i)),
    out_shape=jax.ShapeDtypeStruct((3, N), jnp.float32),
)


def kernel(Z, position, atomic_masses):
    post = position.T
    tab = jnp.zeros((128,), jnp.float32).at[: atomic_masses.shape[0]].set(atomic_masses)
    m = _sc_gather_masses(Z, tab)
    sums = _tc_reduce(post, m)
    outt = _tc_subtract(post, sums)
    return outt.T


# speedup vs baseline: 75.7362x; 1.0960x over previous
import functools

import jax
import jax.numpy as jnp
from jax import lax
from jax.experimental import pallas as pl
from jax.experimental.pallas import tpu as pltpu
from jax.experimental.pallas import tpu_sc as plsc

NC, NS, L = 2, 16, 16
NW = NC * NS
N = 1_000_000
C = 8_000
NCHUNK = N // C
GRP = C // L
FULL = NCHUNK // NW
REM = NCHUNK % NW
MAXC = FULL + 1

_mesh = plsc.VectorSubcoreMesh(core_axis_name="c", subcore_axis_name="s")
_params = pltpu.CompilerParams(needs_layout_passes=False)


@functools.partial(
    pl.kernel,
    out_type=jax.ShapeDtypeStruct((N,), jnp.float32),
    mesh=_mesh,
    compiler_params=_params,
    scratch_types=[
        pltpu.VMEM((128,), jnp.float32),
        [pltpu.VMEM((C,), jnp.int32) for _ in range(MAXC)],
        [pltpu.VMEM((C,), jnp.float32) for _ in range(MAXC)],
        pltpu.SemaphoreType.DMA,
        pltpu.SemaphoreType.DMA,
    ],
)
def _sc_gather_masses(z_hbm, tab_hbm, m_hbm, tab_v, z_v, m_v, zsem, osem):
    wid = lax.axis_index("s") * NC + lax.axis_index("c")
    pltpu.sync_copy(tab_hbm, tab_v)
    has_extra = wid < REM

    copies = []
    for k in range(MAXC):
        c = wid + k * NW
        cp = pltpu.make_async_copy(z_hbm.at[pl.ds(c * C, C)], z_v[k], zsem)
        if k < FULL:
            cp.start()
        else:
            @pl.when(has_extra)
            def _(cp=cp):
                cp.start()
        copies.append(cp)

    out_copies = []
    for k in range(MAXC):
        c = wid + k * NW
        ocp = pltpu.make_async_copy(m_v[k], m_hbm.at[pl.ds(c * C, C)], osem)

        def _do(k=k, cp=copies[k], ocp=ocp):
            cp.wait()

            @plsc.parallel_loop(0, GRP, unroll=8)
            def _(g):
                z = z_v[k][pl.ds(g * L, L)]
                m_v[k][pl.ds(g * L, L)] = plsc.load_gather(tab_v, [z])

            ocp.start()

        if k < FULL:
            _do()
        else:
            pl.when(has_extra)(_do)
        out_copies.append(ocp)

    for k in range(MAXC):
        if k < FULL:
            out_copies[k].wait()
        else:
            @pl.when(has_extra)
            def _(ocp=out_copies[k]):
                ocp.wait()


B = 32_768
NB = -(-N // B)


def _tc_fused_body(pos_ref, m_ref, out_ref, acc_ref, com_ref):
    p = pl.program_id(0)
    i = pl.program_id(1)

    @pl.when(jnp.logical_and(p == 0, i == 0))
    def _():
        acc_ref[...] = jnp.zeros_like(acc_ref)

    @pl.when(p == 0)
    def _():
        lane = lax.broadcasted_iota(jnp.int32, (1, B), 1) + i * B
        m = jnp.where(lane < N, m_ref[...].reshape(1, B), 0.0)
        acc_ref[0:3, :] += m * pos_ref[...]
        acc_ref[3:4, :] += m

    @pl.when(jnp.logical_and(p == 1, i == 0))
    def _():
        s = jnp.sum(acc_ref[...], axis=1, keepdims=True)
        com_ref[0:3, :] = jnp.broadcast_to(s[0:3, :] / s[3:4, :], (3, 128))

    @pl.when(p == 1)
    def _():
        out_ref[...] = pos_ref[...] - com_ref[0:3, 0:1]


_tc_fused = pl.pallas_call(
    _tc_fused_body,
    grid=(2, NB),
    in_specs=[
        pl.BlockSpec((3, B), lambda p, i: (0, i)),
        pl.BlockSpec((B,), lambda p, i: (i * (1 - p),)),
    ],
    out_specs=pl.BlockSpec((3, B), lambda p, i: (0, i * p)),
    out_shape=jax.ShapeDtypeStruct((3, N), jnp.float32),
    scratch_shapes=[
        pltpu.VMEM((4, B), jnp.float32),
        pltpu.VMEM((4, 128), jnp.float32),
    ],
)


def kernel(Z, position, atomic_masses):
    post = position.T
    tab = jnp.zeros((128,), jnp.float32).at[: atomic_masses.shape[0]].set(atomic_masses)
    m = _sc_gather_masses(Z, tab)
    outt = _tc_fused(post, m)
    return outt.T


# speedup vs baseline: 96.3228x; 1.2718x over previous
import functools

import jax
import jax.numpy as jnp
from jax import lax
from jax.experimental import pallas as pl
from jax.experimental.pallas import tpu as pltpu
from jax.experimental.pallas import tpu_sc as plsc

NC, NS, L = 2, 16, 16
NW = NC * NS
N = 1_000_000
C = 8_000
NCHUNK = N // C
GRP = C // L
FULL = NCHUNK // NW
REM = NCHUNK % NW
MAXC = FULL + 1

_mesh = plsc.VectorSubcoreMesh(core_axis_name="c", subcore_axis_name="s")
_params = pltpu.CompilerParams(needs_layout_passes=False)


@functools.partial(
    pl.kernel,
    out_type=jax.ShapeDtypeStruct((N,), jnp.float32),
    mesh=_mesh,
    compiler_params=_params,
    scratch_types=[
        pltpu.VMEM((128,), jnp.float32),
        [pltpu.VMEM((C,), jnp.int32) for _ in range(MAXC)],
        [pltpu.VMEM((C,), jnp.float32) for _ in range(MAXC)],
        pltpu.SemaphoreType.DMA,
        pltpu.SemaphoreType.DMA,
    ],
)
def _sc_gather_masses(z_hbm, tab_hbm, m_hbm, tab_v, z_v, m_v, zsem, osem):
    wid = lax.axis_index("s") * NC + lax.axis_index("c")
    pltpu.sync_copy(tab_hbm, tab_v)
    has_extra = wid < REM

    copies = []
    for k in range(MAXC):
        c = wid + k * NW
        cp = pltpu.make_async_copy(z_hbm.at[pl.ds(c * C, C)], z_v[k], zsem)
        if k < FULL:
            cp.start()
        else:
            @pl.when(has_extra)
            def _(cp=cp):
                cp.start()
        copies.append(cp)

    out_copies = []
    for k in range(MAXC):
        c = wid + k * NW
        ocp = pltpu.make_async_copy(m_v[k], m_hbm.at[pl.ds(c * C, C)], osem)

        def _do(k=k, cp=copies[k], ocp=ocp):
            cp.wait()

            @plsc.parallel_loop(0, GRP, unroll=8)
            def _(g):
                z = z_v[k][pl.ds(g * L, L)]
                m_v[k][pl.ds(g * L, L)] = plsc.load_gather(tab_v, [z])

            ocp.start()

        if k < FULL:
            _do()
        else:
            pl.when(has_extra)(_do)
        out_copies.append(ocp)

    for k in range(MAXC):
        if k < FULL:
            out_copies[k].wait()
        else:
            @pl.when(has_extra)
            def _(ocp=out_copies[k]):
                ocp.wait()


B = 65_536
NB = -(-N // B)
LASTV = N - (NB - 1) * B


def _tc_fused_body(pos_ref, m_ref, out_ref, acc_ref, com_ref):
    p = pl.program_id(0)
    i = pl.program_id(1)

    @pl.when(jnp.logical_and(p == 0, i == 0))
    def _():
        acc_ref[...] = jnp.zeros_like(acc_ref)

    def _accumulate(masked):
        m = m_ref[...].reshape(1, B)
        if masked:
            lane = lax.broadcasted_iota(jnp.int32, (1, B), 1)
            m = jnp.where(lane < LASTV, m, 0.0)
        acc_ref[0:3, :] += m * pos_ref[...]
        acc_ref[3:4, :] += m

    pl.when(jnp.logical_and(p == 0, i < NB - 1))(lambda: _accumulate(False))
    pl.when(jnp.logical_and(p == 0, i == NB - 1))(lambda: _accumulate(True))

    @pl.when(jnp.logical_and(p == 1, i == 0))
    def _():
        s = jnp.sum(acc_ref[...], axis=1, keepdims=True)
        com_ref[0:3, :] = jnp.broadcast_to(s[0:3, :] / s[3:4, :], (3, 128))

    @pl.when(p == 1)
    def _():
        out_ref[...] = pos_ref[...] - com_ref[0:3, 0:1]


_tc_fused = pl.pallas_call(
    _tc_fused_body,
    grid=(2, NB),
    in_specs=[
        pl.BlockSpec((3, B), lambda p, i: (0, i)),
        pl.BlockSpec((B,), lambda p, i: (i * (1 - p),)),
    ],
    out_specs=pl.BlockSpec((3, B), lambda p, i: (0, i * p)),
    out_shape=jax.ShapeDtypeStruct((3, N), jnp.float32),
    scratch_shapes=[
        pltpu.VMEM((4, B), jnp.float32),
        pltpu.VMEM((4, 128), jnp.float32),
    ],
)


def kernel(Z, position, atomic_masses):
    post = position.T
    tab = jnp.zeros((128,), jnp.float32).at[: atomic_masses.shape[0]].set(atomic_masses)
    m = _sc_gather_masses(Z, tab)
    outt = _tc_fused(post, m)
    return outt.T
